# baseline (speedup 1.0000x reference)
import functools

import jax
import jax.numpy as jnp
from jax import lax
from jax.experimental import pallas as pl
from jax.experimental.pallas import tpu as pltpu
from jax.experimental.pallas import tpu_sc as plsc

D = 64
B = 16384

_INFO = plsc.get_sparse_core_info()
NC = _INFO.num_cores
NS = _INFO.num_subcores
L = _INFO.num_lanes
NW = NC * NS
BPW = B // NW
CHUNK = 128
NCHUNK = BPW // CHUNK
NGROUP = BPW // L

_MAGIC = 0x5F3759DF
_INV_EPS = 1e8


def _rsqrt16(x):
    i = plsc.bitcast(x, jnp.int32)
    i = jnp.full((L,), _MAGIC, jnp.int32) - (i >> 1)
    y = plsc.bitcast(i, jnp.float32)
    half_x = x * 0.5
    for _ in range(3):
        y = y * (1.5 - half_x * y * y)
    return jnp.minimum(y, jnp.full((L,), _INV_EPS, jnp.float32))


def _sc_body(uidx_hbm, iidx_hbm, utab_hbm, itab_hbm, out_hbm,
             uidx_v, iidx_v, urows_v, irows_v, out_v, sem):
    wid = lax.axis_index("s") * NC + lax.axis_index("c")
    base = wid * BPW

    for c in range(NCHUNK):
        pltpu.sync_copy(uidx_hbm.at[pl.ds(base + c * CHUNK, CHUNK)],
                        uidx_v.at[c])
        pltpu.sync_copy(iidx_hbm.at[pl.ds(base + c * CHUNK, CHUNK)],
                        iidx_v.at[c])

    copies = []
    for c in range(NCHUNK):
        copies.append(pltpu.async_copy(
            utab_hbm.at[uidx_v.at[c]],
            urows_v.at[pl.ds(c * CHUNK, CHUNK)], sem))
        copies.append(pltpu.async_copy(
            itab_hbm.at[iidx_v.at[c]],
            irows_v.at[pl.ds(c * CHUNK, CHUNK)], sem))
    for cp in copies:
        cp.wait()

    lane = lax.iota(jnp.int32, L)

    def group(g, carry):
        row_idx = lane + g * L
        dot = jnp.zeros((L,), jnp.float32)
        uu = jnp.zeros((L,), jnp.float32)
        vv = jnp.zeros((L,), jnp.float32)
        for k in range(D):
            col = jnp.full((L,), k, jnp.int32)
            u = plsc.load_gather(urows_v, [row_idx, col])
            v = plsc.load_gather(irows_v, [row_idx, col])
            dot = dot + u * v
            uu = uu + u * u
            vv = vv + v * v
        res = dot * _rsqrt16(uu) * _rsqrt16(vv)
        out_v[pl.ds(g * L, L)] = res
        return carry

    lax.fori_loop(0, NGROUP, group, 0)

    pltpu.sync_copy(out_v, out_hbm.at[pl.ds(base, BPW)])


def kernel(user_indices, item_indices, user_table, item_table):
    mesh = plsc.VectorSubcoreMesh(core_axis_name="c", subcore_axis_name="s")
    k = functools.partial(
        pl.kernel,
        mesh=mesh,
        out_type=jax.ShapeDtypeStruct((B,), jnp.float32),
        compiler_params=pltpu.CompilerParams(
            needs_layout_passes=False, use_tc_tiling_on_sc=False),
        scratch_types=[
            pltpu.VMEM((NCHUNK, CHUNK), jnp.int32),
            pltpu.VMEM((NCHUNK, CHUNK), jnp.int32),
            pltpu.VMEM((BPW, D), jnp.float32),
            pltpu.VMEM((BPW, D), jnp.float32),
            pltpu.VMEM((BPW,), jnp.float32),
            pltpu.SemaphoreType.DMA,
        ],
    )(_sc_body)
    return k(user_indices.astype(jnp.int32),
             item_indices.astype(jnp.int32),
             user_table, item_table)
